# baseline (speedup 1.0000x reference)
import jax
import jax.numpy as jnp
import numpy as np
from jax.experimental import pallas as pl
from jax.experimental.pallas import tpu as pltpu

D_MODEL = 1024
D_FF = 4096
E = 8
N = 4096
EPS_COMBINE = float(np.finfo(np.float64).eps)

B_ROWS = 512
FF_BLK = 1024
FT = D_FF // FF_BLK
NPAIRS = 2 * N
NBS = NPAIRS // B_ROWS + E


def _gating_kernel(logits_ref, idx_ref, gate_ref, loss_ref):
    logits = logits_ref[...]
    iota = jax.lax.broadcasted_iota(jnp.int32, logits.shape, 1)
    m1 = jnp.max(logits, axis=1, keepdims=True)
    i1 = jnp.min(jnp.where(logits == m1, iota, E), axis=1, keepdims=True)
    l2 = jnp.where(iota == i1, -jnp.inf, logits)
    m2 = jnp.max(l2, axis=1, keepdims=True)
    i2 = jnp.min(jnp.where(l2 == m2, iota, E), axis=1, keepdims=True)
    e2 = jnp.exp(m2 - m1)
    denom = 1.0 + e2
    g1 = 1.0 / denom
    g2 = e2 / denom
    idx_ref[...] = jnp.concatenate([i1, i2], axis=1)
    gate_ref[...] = jnp.concatenate([g1, g2], axis=1)
    gates_full = jnp.where(iota == i1, g1, 0.0) + jnp.where(iota == i2, g2, 0.0)
    importance = jnp.sum(gates_full, axis=0)
    load = jnp.sum((gates_full > 0.0).astype(jnp.float32), axis=0)

    def cv_sq(v):
        mean = jnp.mean(v)
        var = jnp.sum((v - mean) ** 2) / (E - 1)
        return var / (mean * mean + 1e-10)

    loss = (cv_sq(importance) + cv_sq(load)) * 0.1
    loss_ref[...] = loss * jnp.ones((1, 1), jnp.float32)


def _ffn_kernel(be_ref, tok_ref, gate_ref, x_ref, w1_ref, b1_ref, w2_ref,
                b2_ref, out_ref, xs, acc):
    b = pl.program_id(0)
    f = pl.program_id(1)

    @pl.when(jnp.logical_and(b == 0, f == 0))
    def _init():
        out_ref[...] = jnp.zeros_like(out_ref)

    @pl.when(f == 0)
    def _gather():
        def body(i, _):
            t = tok_ref[b * B_ROWS + i]
            xs[pl.ds(i, 1), :] = x_ref[pl.ds(t, 1), :]
            return 0
        jax.lax.fori_loop(0, B_ROWS, body, 0)
        acc[...] = jnp.zeros_like(acc)

    h = jnp.dot(xs[...], w1_ref[0], preferred_element_type=jnp.float32)
    h = jnp.maximum(h + b1_ref[0], 0.0)
    acc[...] += jnp.dot(h, w2_ref[0], preferred_element_type=jnp.float32)

    @pl.when(f == FT - 1)
    def _scatter():
        acc[...] = (acc[...] + b2_ref[0]) * gate_ref[0]

        def body(i, _):
            t = tok_ref[b * B_ROWS + i]
            out_ref[pl.ds(t, 1), :] += acc[pl.ds(i, 1), :]
            return 0
        jax.lax.fori_loop(0, B_ROWS, body, 0)

    @pl.when(jnp.logical_and(b == NBS - 1, f == FT - 1))
    def _eps():
        out_ref[...] = jnp.where(out_ref[...] == 0.0,
                                 jnp.float32(EPS_COMBINE), out_ref[...])


def _ffn_call(x, W1, b1, W2, b2, block_expert, row_token, row_gate3d):
    grid_spec = pltpu.PrefetchScalarGridSpec(
        num_scalar_prefetch=2,
        grid=(NBS, FT),
        in_specs=[
            pl.BlockSpec((1, B_ROWS, 1), lambda b, f, be, tok: (b, 0, 0)),
            pl.BlockSpec((N, D_MODEL), lambda b, f, be, tok: (0, 0)),
            pl.BlockSpec((1, D_MODEL, FF_BLK),
                         lambda b, f, be, tok: (be[b], 0, f)),
            pl.BlockSpec((1, 1, FF_BLK), lambda b, f, be, tok: (be[b], 0, f)),
            pl.BlockSpec((1, FF_BLK, D_MODEL),
                         lambda b, f, be, tok: (be[b], f, 0)),
            pl.BlockSpec((1, 1, D_MODEL), lambda b, f, be, tok: (be[b], 0, 0)),
        ],
        out_specs=pl.BlockSpec((N, D_MODEL), lambda b, f, be, tok: (0, 0)),
        scratch_shapes=[
            pltpu.VMEM((B_ROWS, D_MODEL), jnp.float32),
            pltpu.VMEM((B_ROWS, D_MODEL), jnp.float32),
        ],
    )
    return pl.pallas_call(
        _ffn_kernel,
        grid_spec=grid_spec,
        out_shape=jax.ShapeDtypeStruct((N, D_MODEL), jnp.float32),
    )(block_expert, row_token, row_gate3d, x,
      W1, b1.reshape(E, 1, D_FF), W2, b2.reshape(E, 1, D_MODEL))


def kernel(x, w_gate, W1, b1, W2, b2):
    logits = x @ w_gate
    idx, gates, loss = pl.pallas_call(
        _gating_kernel,
        out_shape=(
            jax.ShapeDtypeStruct((N, 2), jnp.int32),
            jax.ShapeDtypeStruct((N, 2), jnp.float32),
            jax.ShapeDtypeStruct((1, 1), jnp.float32),
        ),
    )(logits)

    pairs_e = jnp.concatenate([idx[:, 0], idx[:, 1]])
    tok_ids = jnp.arange(N, dtype=jnp.int32)
    pairs_t = jnp.concatenate([tok_ids, tok_ids])
    pairs_g = jnp.concatenate([gates[:, 0], gates[:, 1]])
    order = jnp.argsort(pairs_e)
    se = pairs_e[order]
    counts = jnp.zeros((E,), jnp.int32).at[pairs_e].add(1)
    offsets = jnp.concatenate([jnp.zeros((1,), jnp.int32),
                               jnp.cumsum(counts)[:-1].astype(jnp.int32)])
    padded = ((counts + B_ROWS - 1) // B_ROWS) * B_ROWS
    pad_cum = jnp.cumsum(padded).astype(jnp.int32)
    pad_start = jnp.concatenate([jnp.zeros((1,), jnp.int32), pad_cum[:-1]])
    pos_in_e = jnp.arange(NPAIRS, dtype=jnp.int32) - offsets[se]
    pp = pad_start[se] + pos_in_e
    row_token = jnp.zeros((NBS * B_ROWS,), jnp.int32).at[pp].set(pairs_t[order])
    row_gate = jnp.zeros((NBS * B_ROWS,), jnp.float32).at[pp].set(pairs_g[order])
    block_expert = jnp.minimum(
        jnp.searchsorted(pad_cum, jnp.arange(NBS, dtype=jnp.int32) * B_ROWS,
                         side='right'),
        E - 1).astype(jnp.int32)
    row_gate3d = row_gate.reshape(NBS, B_ROWS, 1)

    y = _ffn_call(x, W1, b1, W2, b2, block_expert, row_token, row_gate3d)
    return (y, loss.reshape(()))


# speedup vs baseline: 1.0483x; 1.0483x over previous
import functools

import jax
import jax.numpy as jnp
import numpy as np
from jax import lax
from jax.experimental import pallas as pl
from jax.experimental.pallas import tpu as pltpu
from jax.experimental.pallas import tpu_sc as plsc

D_MODEL = 1024
D_FF = 4096
E = 8
N = 4096
EPS_COMBINE = float(np.finfo(np.float64).eps)

B_ROWS = 512
FF_BLK = 2048
FT = D_FF // FF_BLK
NPAIRS = 2 * N
NBS = NPAIRS // B_ROWS + E
PAD_ROWS = NBS * B_ROWS

NW = 32
GPW = PAD_ROWS // NW
GCH = 64
TPW = N // NW
CCH = 32
_LANES = 16


def _gating_kernel(logits_ref, idx_ref, gate_ref, loss_ref):
    logits = logits_ref[...]
    iota = jax.lax.broadcasted_iota(jnp.int32, logits.shape, 1)
    m1 = jnp.max(logits, axis=1, keepdims=True)
    i1 = jnp.min(jnp.where(logits == m1, iota, E), axis=1, keepdims=True)
    l2 = jnp.where(iota == i1, -jnp.inf, logits)
    m2 = jnp.max(l2, axis=1, keepdims=True)
    i2 = jnp.min(jnp.where(l2 == m2, iota, E), axis=1, keepdims=True)
    e2 = jnp.exp(m2 - m1)
    denom = 1.0 + e2
    g1 = 1.0 / denom
    g2 = e2 / denom
    idx_ref[...] = jnp.concatenate([i1, i2], axis=1)
    gate_ref[...] = jnp.concatenate([g1, g2], axis=1)
    gates_full = jnp.where(iota == i1, g1, 0.0) + jnp.where(iota == i2, g2, 0.0)
    importance = jnp.sum(gates_full, axis=0)
    load = jnp.sum((gates_full > 0.0).astype(jnp.float32), axis=0)

    def cv_sq(v):
        mean = jnp.mean(v)
        var = jnp.sum((v - mean) ** 2) / (E - 1)
        return var / (mean * mean + 1e-10)

    loss = (cv_sq(importance) + cv_sq(load)) * 0.1
    loss_ref[...] = loss * jnp.ones((1, 1), jnp.float32)


@functools.partial(
    pl.kernel,
    out_type=jax.ShapeDtypeStruct((PAD_ROWS, D_MODEL), jnp.float32),
    mesh=plsc.VectorSubcoreMesh(core_axis_name="c", subcore_axis_name="s"),
    scratch_types=[
        pltpu.VMEM((GPW,), jnp.int32),
        pltpu.VMEM((GCH, D_MODEL), jnp.float32),
        pltpu.SemaphoreType.DMA,
    ],
)
def _sc_gather(x_hbm, tok_hbm, out_hbm, idx_v, buf, sem):
    wid = lax.axis_index("s") * 2 + lax.axis_index("c")
    base = wid * GPW
    pltpu.sync_copy(tok_hbm.at[pl.ds(base, GPW)], idx_v)
    for c in range(GPW // GCH):
        pltpu.async_copy(
            x_hbm.at[idx_v.at[pl.ds(c * GCH, GCH)]], buf, sem).wait()
        pltpu.sync_copy(buf, out_hbm.at[pl.ds(base + c * GCH, GCH)])


@functools.partial(
    pl.kernel,
    out_type=jax.ShapeDtypeStruct((N, D_MODEL), jnp.float32),
    mesh=plsc.VectorSubcoreMesh(core_axis_name="c", subcore_axis_name="s"),
    scratch_types=[
        pltpu.VMEM((TPW,), jnp.int32),
        pltpu.VMEM((TPW,), jnp.int32),
        pltpu.VMEM((CCH, D_MODEL), jnp.float32),
        pltpu.VMEM((CCH, D_MODEL), jnp.float32),
        pltpu.SemaphoreType.DMA,
        pltpu.SemaphoreType.DMA,
    ],
)
def _sc_combine(yp_hbm, q1_hbm, q2_hbm, y_hbm, q1_v, q2_v, r1, r2, s1, s2):
    wid = lax.axis_index("s") * 2 + lax.axis_index("c")
    base = wid * TPW
    pltpu.sync_copy(q1_hbm.at[pl.ds(base, TPW)], q1_v)
    pltpu.sync_copy(q2_hbm.at[pl.ds(base, TPW)], q2_v)
    eps = jnp.float32(EPS_COMBINE)
    for c in range(TPW // CCH):
        cp1 = pltpu.async_copy(
            yp_hbm.at[q1_v.at[pl.ds(c * CCH, CCH)]], r1, s1)
        cp2 = pltpu.async_copy(
            yp_hbm.at[q2_v.at[pl.ds(c * CCH, CCH)]], r2, s2)
        cp1.wait()
        cp2.wait()

        def tok_body(t, _):
            for j in range(D_MODEL // _LANES):
                v = (r1[t, pl.ds(j * _LANES, _LANES)]
                     + r2[t, pl.ds(j * _LANES, _LANES)])
                r1[t, pl.ds(j * _LANES, _LANES)] = jnp.where(v == 0.0, eps, v)
            return 0

        jax.lax.fori_loop(0, CCH, tok_body, 0)
        pltpu.sync_copy(r1, y_hbm.at[pl.ds(base + c * CCH, CCH)])


def _ffn_kernel(be_ref, gate_ref, xs_ref, w1_ref, b1_ref, w2_ref, b2_ref,
                out_ref):
    f = pl.program_id(1)

    @pl.when(f == 0)
    def _init():
        out_ref[...] = jnp.zeros_like(out_ref)

    h = jnp.dot(xs_ref[...], w1_ref[0], preferred_element_type=jnp.float32)
    h = jnp.maximum(h + b1_ref[0], 0.0)
    out_ref[...] += jnp.dot(h, w2_ref[0], preferred_element_type=jnp.float32)

    @pl.when(f == FT - 1)
    def _finish():
        out_ref[...] = (out_ref[...] + b2_ref[0]) * gate_ref[0]


def _ffn_call(xs, W1, b1, W2, b2, block_expert, row_gate3d):
    grid_spec = pltpu.PrefetchScalarGridSpec(
        num_scalar_prefetch=1,
        grid=(NBS, FT),
        in_specs=[
            pl.BlockSpec((1, B_ROWS, 1), lambda b, f, be: (b, 0, 0)),
            pl.BlockSpec((B_ROWS, D_MODEL), lambda b, f, be: (b, 0)),
            pl.BlockSpec((1, D_MODEL, FF_BLK), lambda b, f, be: (be[b], 0, f)),
            pl.BlockSpec((1, 1, FF_BLK), lambda b, f, be: (be[b], 0, f)),
            pl.BlockSpec((1, FF_BLK, D_MODEL), lambda b, f, be: (be[b], f, 0)),
            pl.BlockSpec((1, 1, D_MODEL), lambda b, f, be: (be[b], 0, 0)),
        ],
        out_specs=pl.BlockSpec((B_ROWS, D_MODEL), lambda b, f, be: (b, 0)),
    )
    return pl.pallas_call(
        _ffn_kernel,
        grid_spec=grid_spec,
        out_shape=jax.ShapeDtypeStruct((PAD_ROWS, D_MODEL), jnp.float32),
    )(block_expert, row_gate3d, xs, W1,
      b1.reshape(E, 1, D_FF), W2, b2.reshape(E, 1, D_MODEL))


def kernel(x, w_gate, W1, b1, W2, b2):
    logits = x @ w_gate
    idx, gates, loss = pl.pallas_call(
        _gating_kernel,
        out_shape=(
            jax.ShapeDtypeStruct((N, 2), jnp.int32),
            jax.ShapeDtypeStruct((N, 2), jnp.float32),
            jax.ShapeDtypeStruct((1, 1), jnp.float32),
        ),
    )(logits)

    pairs_e = jnp.concatenate([idx[:, 0], idx[:, 1]])
    tok_ids = jnp.arange(N, dtype=jnp.int32)
    pairs_t = jnp.concatenate([tok_ids, tok_ids])
    pairs_g = jnp.concatenate([gates[:, 0], gates[:, 1]])
    order = jnp.argsort(pairs_e)
    se = pairs_e[order]
    counts = jnp.zeros((E,), jnp.int32).at[pairs_e].add(1)
    offsets = jnp.concatenate([jnp.zeros((1,), jnp.int32),
                               jnp.cumsum(counts)[:-1].astype(jnp.int32)])
    padded = ((counts + B_ROWS - 1) // B_ROWS) * B_ROWS
    pad_cum = jnp.cumsum(padded).astype(jnp.int32)
    pad_start = jnp.concatenate([jnp.zeros((1,), jnp.int32), pad_cum[:-1]])
    pos_in_e = jnp.arange(NPAIRS, dtype=jnp.int32) - offsets[se]
    pp = pad_start[se] + pos_in_e
    row_token = jnp.zeros((PAD_ROWS,), jnp.int32).at[pp].set(pairs_t[order])
    row_gate = jnp.zeros((PAD_ROWS,), jnp.float32).at[pp].set(pairs_g[order])
    block_expert = jnp.minimum(
        jnp.searchsorted(pad_cum, jnp.arange(NBS, dtype=jnp.int32) * B_ROWS,
                         side='right'),
        E - 1).astype(jnp.int32)
    inv = jnp.zeros((NPAIRS,), jnp.int32).at[order].set(
        jnp.arange(NPAIRS, dtype=jnp.int32))
    qq = pp[inv]
    q1 = qq[:N]
    q2 = qq[N:]
    row_gate3d = row_gate.reshape(NBS, B_ROWS, 1)

    xs = _sc_gather(x, row_token)
    yp = _ffn_call(xs, W1, b1, W2, b2, block_expert, row_gate3d)
    y = _sc_combine(yp, q1, q2)
    return (y, loss.reshape(()))


# speedup vs baseline: 1.0502x; 1.0019x over previous
import functools

import jax
import jax.numpy as jnp
import numpy as np
from jax import lax
from jax.experimental import pallas as pl
from jax.experimental.pallas import tpu as pltpu
from jax.experimental.pallas import tpu_sc as plsc

D_MODEL = 1024
D_FF = 4096
E = 8
N = 4096
EPS_COMBINE = float(np.finfo(np.float64).eps)

B_ROWS = 512
FF_BLK = 2048
FT = D_FF // FF_BLK
NPAIRS = 2 * N
NBS = NPAIRS // B_ROWS + E
PAD_ROWS = NBS * B_ROWS

NW = 32
GPW = PAD_ROWS // NW
GCH = 32
TPW = N // NW
CCH = 32
_LANES = 16


def _gating_kernel(logits_ref, idx_ref, gate_ref, loss_ref):
    logits = logits_ref[...]
    iota = jax.lax.broadcasted_iota(jnp.int32, logits.shape, 1)
    m1 = jnp.max(logits, axis=1, keepdims=True)
    i1 = jnp.min(jnp.where(logits == m1, iota, E), axis=1, keepdims=True)
    l2 = jnp.where(iota == i1, -jnp.inf, logits)
    m2 = jnp.max(l2, axis=1, keepdims=True)
    i2 = jnp.min(jnp.where(l2 == m2, iota, E), axis=1, keepdims=True)
    e2 = jnp.exp(m2 - m1)
    denom = 1.0 + e2
    g1 = 1.0 / denom
    g2 = e2 / denom
    idx_ref[...] = jnp.concatenate([i1, i2], axis=1)
    gate_ref[...] = jnp.concatenate([g1, g2], axis=1)
    gates_full = jnp.where(iota == i1, g1, 0.0) + jnp.where(iota == i2, g2, 0.0)
    importance = jnp.sum(gates_full, axis=0)
    load = jnp.sum((gates_full > 0.0).astype(jnp.float32), axis=0)

    def cv_sq(v):
        mean = jnp.mean(v)
        var = jnp.sum((v - mean) ** 2) / (E - 1)
        return var / (mean * mean + 1e-10)

    loss = (cv_sq(importance) + cv_sq(load)) * 0.1
    loss_ref[...] = loss * jnp.ones((1, 1), jnp.float32)


@functools.partial(
    pl.kernel,
    out_type=jax.ShapeDtypeStruct((PAD_ROWS, D_MODEL), jnp.float32),
    mesh=plsc.VectorSubcoreMesh(core_axis_name="c", subcore_axis_name="s"),
    scratch_types=[
        pltpu.VMEM((GPW,), jnp.int32),
        pltpu.VMEM((GCH, D_MODEL), jnp.float32),
        pltpu.VMEM((GCH, D_MODEL), jnp.float32),
        pltpu.SemaphoreType.DMA,
        pltpu.SemaphoreType.DMA,
    ],
)
def _sc_gather(x_hbm, tok_hbm, out_hbm, idx_v, buf0, buf1, sem0, sem1):
    wid = lax.axis_index("s") * 2 + lax.axis_index("c")
    base = wid * GPW
    pltpu.sync_copy(tok_hbm.at[pl.ds(base, GPW)], idx_v)
    bufs = (buf0, buf1)
    sems = (sem0, sem1)
    nch = GPW // GCH
    cp = pltpu.async_copy(x_hbm.at[idx_v.at[pl.ds(0, GCH)]], buf0, sem0)
    for c in range(nch):
        b = c & 1
        nxt = None
        if c + 1 < nch:
            nxt = pltpu.async_copy(
                x_hbm.at[idx_v.at[pl.ds((c + 1) * GCH, GCH)]],
                bufs[1 - b], sems[1 - b])
        cp.wait()
        pltpu.sync_copy(bufs[b], out_hbm.at[pl.ds(base + c * GCH, GCH)])
        cp = nxt


@functools.partial(
    pl.kernel,
    out_type=jax.ShapeDtypeStruct((N, D_MODEL), jnp.float32),
    mesh=plsc.VectorSubcoreMesh(core_axis_name="c", subcore_axis_name="s"),
    scratch_types=[
        pltpu.VMEM((TPW,), jnp.int32),
        pltpu.VMEM((TPW,), jnp.int32),
        pltpu.VMEM((CCH, D_MODEL), jnp.float32),
        pltpu.VMEM((CCH, D_MODEL), jnp.float32),
        pltpu.SemaphoreType.DMA,
        pltpu.SemaphoreType.DMA,
    ],
)
def _sc_combine(yp_hbm, q1_hbm, q2_hbm, y_hbm, q1_v, q2_v, r1, r2, s1, s2):
    wid = lax.axis_index("s") * 2 + lax.axis_index("c")
    base = wid * TPW
    pltpu.sync_copy(q1_hbm.at[pl.ds(base, TPW)], q1_v)
    pltpu.sync_copy(q2_hbm.at[pl.ds(base, TPW)], q2_v)
    eps = jnp.float32(EPS_COMBINE)
    for c in range(TPW // CCH):
        cp1 = pltpu.async_copy(
            yp_hbm.at[q1_v.at[pl.ds(c * CCH, CCH)]], r1, s1)
        cp2 = pltpu.async_copy(
            yp_hbm.at[q2_v.at[pl.ds(c * CCH, CCH)]], r2, s2)
        cp1.wait()
        cp2.wait()

        def tok_body(t, _):
            for j in range(D_MODEL // _LANES):
                v = (r1[t, pl.ds(j * _LANES, _LANES)]
                     + r2[t, pl.ds(j * _LANES, _LANES)])
                r1[t, pl.ds(j * _LANES, _LANES)] = jnp.where(v == 0.0, eps, v)
            return 0

        jax.lax.fori_loop(0, CCH, tok_body, 0)
        pltpu.sync_copy(r1, y_hbm.at[pl.ds(base + c * CCH, CCH)])


def _ffn_kernel(be_ref, gate_ref, xs_ref, w1_ref, b1_ref, w2_ref, b2_ref,
                out_ref):
    f = pl.program_id(1)

    @pl.when(f == 0)
    def _init():
        out_ref[...] = jnp.zeros_like(out_ref)

    h = jnp.dot(xs_ref[...], w1_ref[0], preferred_element_type=jnp.float32)
    h = jnp.maximum(h + b1_ref[0], 0.0)
    out_ref[...] += jnp.dot(h, w2_ref[0], preferred_element_type=jnp.float32)

    @pl.when(f == FT - 1)
    def _finish():
        out_ref[...] = (out_ref[...] + b2_ref[0]) * gate_ref[0]


def _ffn_call(xs, W1, b1, W2, b2, block_expert, row_gate3d):
    grid_spec = pltpu.PrefetchScalarGridSpec(
        num_scalar_prefetch=1,
        grid=(NBS, FT),
        in_specs=[
            pl.BlockSpec((1, B_ROWS, 1), lambda b, f, be: (b, 0, 0)),
            pl.BlockSpec((B_ROWS, D_MODEL), lambda b, f, be: (b, 0)),
            pl.BlockSpec((1, D_MODEL, FF_BLK), lambda b, f, be: (be[b], 0, f)),
            pl.BlockSpec((1, 1, FF_BLK), lambda b, f, be: (be[b], 0, f)),
            pl.BlockSpec((1, FF_BLK, D_MODEL), lambda b, f, be: (be[b], f, 0)),
            pl.BlockSpec((1, 1, D_MODEL), lambda b, f, be: (be[b], 0, 0)),
        ],
        out_specs=pl.BlockSpec((B_ROWS, D_MODEL), lambda b, f, be: (b, 0)),
    )
    return pl.pallas_call(
        _ffn_kernel,
        grid_spec=grid_spec,
        out_shape=jax.ShapeDtypeStruct((PAD_ROWS, D_MODEL), jnp.float32),
    )(block_expert, row_gate3d, xs, W1,
      b1.reshape(E, 1, D_FF), W2, b2.reshape(E, 1, D_MODEL))


def kernel(x, w_gate, W1, b1, W2, b2):
    logits = x @ w_gate
    idx, gates, loss = pl.pallas_call(
        _gating_kernel,
        out_shape=(
            jax.ShapeDtypeStruct((N, 2), jnp.int32),
            jax.ShapeDtypeStruct((N, 2), jnp.float32),
            jax.ShapeDtypeStruct((1, 1), jnp.float32),
        ),
    )(logits)

    pairs_e = jnp.concatenate([idx[:, 0], idx[:, 1]])
    tok_ids = jnp.arange(N, dtype=jnp.int32)
    pairs_t = jnp.concatenate([tok_ids, tok_ids])
    pairs_g = jnp.concatenate([gates[:, 0], gates[:, 1]])
    order = jnp.argsort(pairs_e)
    se = pairs_e[order]
    counts = jnp.zeros((E,), jnp.int32).at[pairs_e].add(1)
    offsets = jnp.concatenate([jnp.zeros((1,), jnp.int32),
                               jnp.cumsum(counts)[:-1].astype(jnp.int32)])
    padded = ((counts + B_ROWS - 1) // B_ROWS) * B_ROWS
    pad_cum = jnp.cumsum(padded).astype(jnp.int32)
    pad_start = jnp.concatenate([jnp.zeros((1,), jnp.int32), pad_cum[:-1]])
    pos_in_e = jnp.arange(NPAIRS, dtype=jnp.int32) - offsets[se]
    pp = pad_start[se] + pos_in_e
    row_token = jnp.zeros((PAD_ROWS,), jnp.int32).at[pp].set(pairs_t[order])
    row_gate = jnp.zeros((PAD_ROWS,), jnp.float32).at[pp].set(pairs_g[order])
    block_expert = jnp.minimum(
        jnp.searchsorted(pad_cum, jnp.arange(NBS, dtype=jnp.int32) * B_ROWS,
                         side='right'),
        E - 1).astype(jnp.int32)
    inv = jnp.zeros((NPAIRS,), jnp.int32).at[order].set(
        jnp.arange(NPAIRS, dtype=jnp.int32))
    qq = pp[inv]
    q1 = qq[:N]
    q2 = qq[N:]
    row_gate3d = row_gate.reshape(NBS, B_ROWS, 1)

    xs = _sc_gather(x, row_token)
    yp = _ffn_call(xs, W1, b1, W2, b2, block_expert, row_gate3d)
    y = _sc_combine(yp, q1, q2)
    return (y, loss.reshape(()))
